# baseline (speedup 1.0000x reference)
import jax
import jax.numpy as jnp
from jax.experimental import pallas as pl
from jax.experimental.pallas import tpu as pltpu


def _body(x_ref, w_ref, sc_ref, sh_ref, o_ref):
    xb = x_ref[0].astype(jnp.bfloat16)
    C, Ht, W = xb.shape
    x5 = xb.reshape(C, Ht // 2, 2, W // 2, 2)
    L = (Ht // 2) * (W // 2)
    acc = None
    for hi in range(2):
        for wi in range(2):
            xc = x5[:, :, hi, :, wi].reshape(C, L)
            g = jax.lax.dot_general(
                w_ref[hi, wi], xc,
                (((1,), (0,)), ((), ())),
                preferred_element_type=jnp.float32)
            acc = g if acc is None else acc + g
    y = acc * sc_ref[...] + sh_ref[...]
    o_ref[0] = jnp.maximum(y, 0.0)


def kernel(x, conv_w, conv_b, bn_gamma, bn_beta, bn_mean, bn_var,
           eps=1e-5):
    N, C, H, W = x.shape
    assert H % 2 == 0 and W % 2 == 0
    H2, W2 = H // 2, W // 2
    out_ch = conv_w.shape[0]

    S = 0.5 * jnp.array([[1.,  1.,  1.,  1.],
                         [1.,  1., -1., -1.],
                         [1., -1.,  1., -1.],
                         [1., -1., -1.,  1.]], dtype=jnp.float32)
    w4 = conv_w.reshape(out_ch, 4, C).astype(jnp.float32)
    wc = jnp.einsum('bk,obc->koc', S, w4).reshape(2, 2, out_ch, C)

    scale = bn_gamma.astype(jnp.float32) / jnp.sqrt(
        bn_var.astype(jnp.float32) + eps)
    shift = (conv_b.astype(jnp.float32) - bn_mean.astype(jnp.float32)) \
        * scale + bn_beta.astype(jnp.float32)

    ocp = ((out_ch + 127) // 128) * 128
    pad = ocp - out_ch
    wc = jnp.pad(wc, ((0, 0), (0, 0), (0, pad), (0, 0))).astype(jnp.bfloat16)
    scale = jnp.pad(scale, (0, pad)).reshape(ocp, 1)
    shift = jnp.pad(shift, (0, pad)).reshape(ocp, 1)

    Ht = 32 if H % 32 == 0 else 2
    T = H // Ht
    L = (Ht // 2) * W2

    out_flat = pl.pallas_call(
        _body,
        out_shape=jax.ShapeDtypeStruct((N, ocp, H2 * W2), x.dtype),
        grid=(N, T),
        in_specs=[
            pl.BlockSpec((1, C, Ht, W), lambda n, t: (n, 0, t, 0)),
            pl.BlockSpec((2, 2, ocp, C), lambda n, t: (0, 0, 0, 0)),
            pl.BlockSpec((ocp, 1), lambda n, t: (0, 0)),
            pl.BlockSpec((ocp, 1), lambda n, t: (0, 0)),
        ],
        out_specs=pl.BlockSpec((1, ocp, L), lambda n, t: (n, 0, t)),
        compiler_params=pltpu.CompilerParams(
            dimension_semantics=("parallel", "parallel"),
            vmem_limit_bytes=64 << 20),
    )(x, wc, scale, shift)

    out = out_flat[:, :out_ch, :].reshape(N, out_ch, H2, W2)
    return out


# speedup vs baseline: 7.4510x; 7.4510x over previous
import jax
import jax.numpy as jnp
from jax.experimental import pallas as pl
from jax.experimental.pallas import tpu as pltpu


def _body(x_ref, a_ref, e_ref, sc_ref, sh_ref, o_ref):
    _, C, Ht2, _, W = x_ref.shape
    OC2 = a_ref.shape[1]
    OC = OC2 // 2
    P = Ht2 // 2
    acc = None
    for hi in range(2):
        xh = x_ref[0, :, :, hi, :].astype(jnp.bfloat16).reshape(C, Ht2 * W)
        g = jax.lax.dot_general(
            a_ref[hi], xh, (((1,), (0,)), ((), ())),
            preferred_element_type=jnp.float32).astype(jnp.bfloat16)
        for wi in range(2):
            lhs = g[wi * OC:(wi + 1) * OC].reshape(OC * P, 2 * W)
            d = jax.lax.dot_general(
                lhs, e_ref[wi], (((1,), (0,)), ((), ())),
                preferred_element_type=jnp.float32)
            acc = d if acc is None else acc + d
    y = acc * sc_ref[...] + sh_ref[...]
    o_ref[0] = jnp.maximum(y, 0.0).reshape(OC, P * W)


def kernel(x, conv_w, conv_b, bn_gamma, bn_beta, bn_mean, bn_var,
           eps=1e-5):
    N, C, H, W = x.shape
    assert H % 2 == 0 and W == 128, "specialized to W=128 lane tiles"
    H2, W2 = H // 2, W // 2
    out_ch = conv_w.shape[0]

    S = 0.5 * jnp.array([[1.,  1.,  1.,  1.],
                         [1.,  1., -1., -1.],
                         [1., -1.,  1., -1.],
                         [1., -1., -1.,  1.]], dtype=jnp.float32)
    w4 = conv_w.reshape(out_ch, 4, C).astype(jnp.float32)
    wc = jnp.einsum('bk,obc->koc', S, w4).reshape(2, 2, out_ch, C)

    scale = bn_gamma.astype(jnp.float32) / jnp.sqrt(
        bn_var.astype(jnp.float32) + eps)
    shift = (conv_b.astype(jnp.float32) - bn_mean.astype(jnp.float32)) \
        * scale + bn_beta.astype(jnp.float32)

    ocp = ((out_ch + 127) // 128) * 128
    pad = ocp - out_ch
    wc = jnp.pad(wc, ((0, 0), (0, 0), (0, pad), (0, 0)))
    scale = jnp.pad(scale, (0, pad))
    shift = jnp.pad(shift, (0, pad))

    A = wc.reshape(2, 2 * ocp, C).astype(jnp.bfloat16)

    q = jnp.arange(2)[:, None]
    w2i = jnp.arange(W2)[None, :]
    rows = (q * W + 2 * w2i).reshape(-1)
    cols = (q * W2 + w2i).reshape(-1)
    E0 = jnp.zeros((2 * W, W), jnp.float32).at[rows, cols].set(1.0)
    E1 = jnp.zeros((2 * W, W), jnp.float32).at[rows + 1, cols].set(1.0)
    E = jnp.stack([E0, E1]).astype(jnp.bfloat16)

    Ht2 = 4 if H2 % 4 == 0 else 2
    T = H2 // Ht2
    P = Ht2 // 2
    L = P * W

    sc_mat = jnp.broadcast_to(jnp.repeat(scale, P)[:, None], (ocp * P, W))
    sh_mat = jnp.broadcast_to(jnp.repeat(shift, P)[:, None], (ocp * P, W))

    x5 = x.reshape(N, C, H2, 2, W)

    out_flat = pl.pallas_call(
        _body,
        out_shape=jax.ShapeDtypeStruct((N, ocp, H2 * W2), x.dtype),
        grid=(N, T),
        in_specs=[
            pl.BlockSpec((1, C, Ht2, 2, W), lambda n, t: (n, 0, t, 0, 0)),
            pl.BlockSpec((2, 2 * ocp, C), lambda n, t: (0, 0, 0)),
            pl.BlockSpec((2, 2 * W, W), lambda n, t: (0, 0, 0)),
            pl.BlockSpec((ocp * P, W), lambda n, t: (0, 0)),
            pl.BlockSpec((ocp * P, W), lambda n, t: (0, 0)),
        ],
        out_specs=pl.BlockSpec((1, ocp, L), lambda n, t: (n, 0, t)),
        compiler_params=pltpu.CompilerParams(
            dimension_semantics=("parallel", "parallel"),
            vmem_limit_bytes=64 << 20),
    )(x5, A, E, sc_mat, sh_mat)

    out = out_flat[:, :out_ch, :].reshape(N, out_ch, H2, W2)
    return out


# speedup vs baseline: 15.4547x; 2.0742x over previous
import jax
import jax.numpy as jnp
from jax.experimental import pallas as pl
from jax.experimental.pallas import tpu as pltpu


def _body(x_ref, a_ref, e_ref, sc_ref, sh_ref, o_ref):
    _, C, Bt, _, W = x_ref.shape
    OC = a_ref.shape[1] // 2
    for r in range(Bt):
        xr = x_ref[0, :, r, :, :].astype(jnp.bfloat16).reshape(C, 8 * W)
        for pl_ in range(2):
            acc = None
            for hi in range(2):
                t0 = (4 * pl_ + hi) * W
                t1 = (4 * pl_ + 2 + hi) * W
                xc = jnp.concatenate(
                    [xr[:, t0:t0 + W], xr[:, t1:t1 + W]], axis=1)
                g = jax.lax.dot_general(
                    a_ref[hi], xc, (((1,), (0,)), ((), ())),
                    preferred_element_type=jnp.float32
                ).astype(jnp.bfloat16)
                for wi in range(2):
                    d = jax.lax.dot_general(
                        g[wi * OC:(wi + 1) * OC], e_ref[wi],
                        (((1,), (0,)), ((), ())),
                        preferred_element_type=jnp.float32)
                    acc = d if acc is None else acc + d
            y = jnp.maximum(acc * sc_ref[...] + sh_ref[...], 0.0)
            col = (2 * r + pl_) * W
            o_ref[0, :, col:col + W] = y


def kernel(x, conv_w, conv_b, bn_gamma, bn_beta, bn_mean, bn_var,
           eps=1e-5):
    N, C, H, W = x.shape
    assert H % 8 == 0 and W == 128, "specialized to W=128 lane tiles"
    H2, W2 = H // 2, W // 2
    out_ch = conv_w.shape[0]

    S = 0.5 * jnp.array([[1.,  1.,  1.,  1.],
                         [1.,  1., -1., -1.],
                         [1., -1.,  1., -1.],
                         [1., -1., -1.,  1.]], dtype=jnp.float32)
    w4 = conv_w.reshape(out_ch, 4, C).astype(jnp.float32)
    wc = jnp.einsum('bk,obc->koc', S, w4).reshape(2, 2, out_ch, C)

    scale = bn_gamma.astype(jnp.float32) / jnp.sqrt(
        bn_var.astype(jnp.float32) + eps)
    shift = (conv_b.astype(jnp.float32) - bn_mean.astype(jnp.float32)) \
        * scale + bn_beta.astype(jnp.float32)

    ocp = ((out_ch + 127) // 128) * 128
    pad = ocp - out_ch
    wc = jnp.pad(wc, ((0, 0), (0, 0), (0, pad), (0, 0)))
    scale = jnp.pad(scale, (0, pad))
    shift = jnp.pad(shift, (0, pad))

    A = wc.reshape(2, 2 * ocp, C).astype(jnp.bfloat16)

    q = jnp.arange(2)[:, None]
    w2i = jnp.arange(W2)[None, :]
    rows = (q * W + 2 * w2i).reshape(-1)
    cols = (q * W2 + w2i).reshape(-1)
    E0 = jnp.zeros((2 * W, W), jnp.float32).at[rows, cols].set(1.0)
    E1 = jnp.zeros((2 * W, W), jnp.float32).at[rows + 1, cols].set(1.0)
    E = jnp.stack([E0, E1]).astype(jnp.bfloat16)

    sc_mat = jnp.broadcast_to(scale[:, None], (ocp, W))
    sh_mat = jnp.broadcast_to(shift[:, None], (ocp, W))

    Bt = 4 if (H // 8) % 4 == 0 else 1
    T = H // 8 // Bt
    L = Bt * 2 * W

    xv = x.reshape(N, C, H // 8, 8, W)

    out_flat = pl.pallas_call(
        _body,
        out_shape=jax.ShapeDtypeStruct((N, ocp, H2 * W2), x.dtype),
        grid=(N, T),
        in_specs=[
            pl.BlockSpec((1, C, Bt, 8, W), lambda n, t: (n, 0, t, 0, 0)),
            pl.BlockSpec((2, 2 * ocp, C), lambda n, t: (0, 0, 0)),
            pl.BlockSpec((2, 2 * W, W), lambda n, t: (0, 0, 0)),
            pl.BlockSpec((ocp, W), lambda n, t: (0, 0)),
            pl.BlockSpec((ocp, W), lambda n, t: (0, 0)),
        ],
        out_specs=pl.BlockSpec((1, ocp, L), lambda n, t: (n, 0, t)),
        compiler_params=pltpu.CompilerParams(
            dimension_semantics=("parallel", "parallel"),
            vmem_limit_bytes=64 << 20),
    )(xv, A, E, sc_mat, sh_mat)

    out = out_flat[:, :out_ch, :].reshape(N, out_ch, H2, W2)
    return out


# speedup vs baseline: 17.0793x; 1.1051x over previous
import numpy as np
import jax
import jax.numpy as jnp
from jax.experimental import pallas as pl
from jax.experimental.pallas import tpu as pltpu


def _body(x_ref, a_ref, e_ref, sc_ref, sh_ref, o_ref):
    _, C, Bt, _, W = x_ref.shape
    OC = a_ref.shape[1] // 2
    for r in range(Bt):
        xr = x_ref[0, :, r, :, :].astype(jnp.bfloat16).reshape(C, 8 * W)
        for pl_ in range(2):
            acc = None
            for hi in range(2):
                t0 = (4 * pl_ + hi) * W
                t1 = (4 * pl_ + 2 + hi) * W
                xc = jnp.concatenate(
                    [xr[:, t0:t0 + W], xr[:, t1:t1 + W]], axis=1)
                g = jax.lax.dot_general(
                    a_ref[hi], xc, (((1,), (0,)), ((), ())),
                    preferred_element_type=jnp.float32
                ).astype(jnp.bfloat16)
                for wi in range(2):
                    d = jax.lax.dot_general(
                        g[wi * OC:(wi + 1) * OC], e_ref[wi],
                        (((1,), (0,)), ((), ())),
                        preferred_element_type=jnp.float32)
                    acc = d if acc is None else acc + d
            y = jnp.maximum(acc * sc_ref[...] + sh_ref[...], 0.0)
            col = (2 * r + pl_) * W
            o_ref[0, :, col:col + W] = y


def kernel(x, conv_w, conv_b, bn_gamma, bn_beta, bn_mean, bn_var,
           eps=1e-5):
    N, C, H, W = x.shape
    assert H % 8 == 0 and W == 128, "specialized to W=128 lane tiles"
    H2, W2 = H // 2, W // 2
    out_ch = conv_w.shape[0]

    S = 0.5 * jnp.array([[1.,  1.,  1.,  1.],
                         [1.,  1., -1., -1.],
                         [1., -1.,  1., -1.],
                         [1., -1., -1.,  1.]], dtype=jnp.float32)
    w4 = conv_w.reshape(out_ch, 4, C).astype(jnp.float32)
    wc = jnp.einsum('bk,obc->koc', S, w4).reshape(2, 2, out_ch, C)

    scale = bn_gamma.astype(jnp.float32) / jnp.sqrt(
        bn_var.astype(jnp.float32) + eps)
    shift = (conv_b.astype(jnp.float32) - bn_mean.astype(jnp.float32)) \
        * scale + bn_beta.astype(jnp.float32)

    ocp = ((out_ch + 127) // 128) * 128
    pad = ocp - out_ch
    wc = jnp.pad(wc, ((0, 0), (0, 0), (0, pad), (0, 0)))
    scale = jnp.pad(scale, (0, pad))
    shift = jnp.pad(shift, (0, pad))

    A = wc.reshape(2, 2 * ocp, C).astype(jnp.bfloat16)

    En = np.zeros((2, 2 * W, W), np.float32)
    for q in range(2):
        for w2 in range(W2):
            En[0, q * W + 2 * w2, q * W2 + w2] = 1.0
            En[1, q * W + 2 * w2 + 1, q * W2 + w2] = 1.0
    E = jnp.asarray(En, dtype=jnp.bfloat16)

    sc_mat = jnp.broadcast_to(scale[:, None], (ocp, W))
    sh_mat = jnp.broadcast_to(shift[:, None], (ocp, W))

    Bt = 4 if (H // 8) % 4 == 0 else 1
    T = H // 8 // Bt
    L = Bt * 2 * W

    xv = x.reshape(N, C, H // 8, 8, W)

    out_flat = pl.pallas_call(
        _body,
        out_shape=jax.ShapeDtypeStruct((N, ocp, H2 * W2), x.dtype),
        grid=(N, T),
        in_specs=[
            pl.BlockSpec((1, C, Bt, 8, W), lambda n, t: (n, 0, t, 0, 0)),
            pl.BlockSpec((2, 2 * ocp, C), lambda n, t: (0, 0, 0)),
            pl.BlockSpec((2, 2 * W, W), lambda n, t: (0, 0, 0)),
            pl.BlockSpec((ocp, W), lambda n, t: (0, 0)),
            pl.BlockSpec((ocp, W), lambda n, t: (0, 0)),
        ],
        out_specs=pl.BlockSpec((1, ocp, L), lambda n, t: (n, 0, t)),
        compiler_params=pltpu.CompilerParams(
            dimension_semantics=("parallel", "parallel"),
            vmem_limit_bytes=64 << 20),
    )(xv, A, E, sc_mat, sh_mat)

    out = out_flat[:, :out_ch, :].reshape(N, out_ch, H2, W2)
    return out


# speedup vs baseline: 20.2204x; 1.1839x over previous
import numpy as np
import jax
import jax.numpy as jnp
from jax.experimental import pallas as pl
from jax.experimental.pallas import tpu as pltpu


def _body(x_ref, a_ref, e_ref, sc_ref, sh_ref, o_ref):
    _, C, Bt, _, W = x_ref.shape
    OC = a_ref.shape[0] // 2
    for r in range(Bt):
        xr = x_ref[0, :, r, :, :].astype(jnp.bfloat16).reshape(C, 8 * W)
        for pl_ in range(2):
            t = [(4 * pl_ + 2 * q + hi) * W for hi in range(2)
                 for q in range(2)]
            rhs = jnp.concatenate(
                [jnp.concatenate([xr[:, t[0]:t[0] + W],
                                  xr[:, t[1]:t[1] + W]], axis=1),
                 jnp.concatenate([xr[:, t[2]:t[2] + W],
                                  xr[:, t[3]:t[3] + W]], axis=1)],
                axis=0)
            g = jax.lax.dot_general(
                a_ref[...], rhs, (((1,), (0,)), ((), ())),
                preferred_element_type=jnp.float32
            ).astype(jnp.bfloat16)
            lhs = jnp.concatenate([g[:OC], g[OC:]], axis=1)
            d = jax.lax.dot_general(
                lhs, e_ref[...], (((1,), (0,)), ((), ())),
                preferred_element_type=jnp.float32)
            y = jnp.maximum(d * sc_ref[...] + sh_ref[...], 0.0)
            col = (2 * r + pl_) * W
            o_ref[0, :, col:col + W] = y


def kernel(x, conv_w, conv_b, bn_gamma, bn_beta, bn_mean, bn_var,
           eps=1e-5):
    N, C, H, W = x.shape
    assert H % 8 == 0 and W == 128, "specialized to W=128 lane tiles"
    H2, W2 = H // 2, W // 2
    out_ch = conv_w.shape[0]

    S = 0.5 * jnp.array([[1.,  1.,  1.,  1.],
                         [1.,  1., -1., -1.],
                         [1., -1.,  1., -1.],
                         [1., -1., -1.,  1.]], dtype=jnp.float32)
    w4 = conv_w.reshape(out_ch, 4, C).astype(jnp.float32)
    wc = jnp.einsum('bk,obc->koc', S, w4).reshape(2, 2, out_ch, C)

    scale = bn_gamma.astype(jnp.float32) / jnp.sqrt(
        bn_var.astype(jnp.float32) + eps)
    shift = (conv_b.astype(jnp.float32) - bn_mean.astype(jnp.float32)) \
        * scale + bn_beta.astype(jnp.float32)

    ocp = ((out_ch + 127) // 128) * 128
    pad = ocp - out_ch
    wc = jnp.pad(wc, ((0, 0), (0, 0), (0, pad), (0, 0)))
    scale = jnp.pad(scale, (0, pad))
    shift = jnp.pad(shift, (0, pad))

    A = wc.transpose(1, 2, 0, 3).reshape(2 * ocp, 2 * C) \
        .astype(jnp.bfloat16)

    En = np.zeros((2, 2 * W, W), np.float32)
    for q in range(2):
        for w2 in range(W2):
            En[0, q * W + 2 * w2, q * W2 + w2] = 1.0
            En[1, q * W + 2 * w2 + 1, q * W2 + w2] = 1.0
    E = jnp.asarray(En.reshape(4 * W, W), dtype=jnp.bfloat16)

    sc_mat = jnp.broadcast_to(scale[:, None], (ocp, W))
    sh_mat = jnp.broadcast_to(shift[:, None], (ocp, W))

    Bt = 4 if (H // 8) % 4 == 0 else 1
    T = H // 8 // Bt
    L = Bt * 2 * W

    xv = x.reshape(N, C, H // 8, 8, W)

    out_flat = pl.pallas_call(
        _body,
        out_shape=jax.ShapeDtypeStruct((N, ocp, H2 * W2), x.dtype),
        grid=(N, T),
        in_specs=[
            pl.BlockSpec((1, C, Bt, 8, W), lambda n, t: (n, 0, t, 0, 0)),
            pl.BlockSpec((2 * ocp, 2 * C), lambda n, t: (0, 0)),
            pl.BlockSpec((4 * W, W), lambda n, t: (0, 0)),
            pl.BlockSpec((ocp, W), lambda n, t: (0, 0)),
            pl.BlockSpec((ocp, W), lambda n, t: (0, 0)),
        ],
        out_specs=pl.BlockSpec((1, ocp, L), lambda n, t: (n, 0, t)),
        compiler_params=pltpu.CompilerParams(
            dimension_semantics=("parallel", "parallel"),
            vmem_limit_bytes=64 << 20),
    )(xv, A, E, sc_mat, sh_mat)

    out = out_flat[:, :out_ch, :].reshape(N, out_ch, H2, W2)
    return out


# speedup vs baseline: 20.2404x; 1.0010x over previous
import numpy as np
import jax
import jax.numpy as jnp
from jax.experimental import pallas as pl
from jax.experimental.pallas import tpu as pltpu


def _body(x_ref, a_ref, e_ref, sc_ref, sh_ref, o_ref):
    _, C, Ht, W = x_ref.shape
    Bt = Ht // 8
    OC = a_ref.shape[0] // 2
    for r in range(Bt):
        xr = x_ref[0, :, 8 * r:8 * (r + 1), :] \
            .astype(jnp.bfloat16).reshape(C, 8 * W)
        for pl_ in range(2):
            t = [(4 * pl_ + 2 * q + hi) * W for hi in range(2)
                 for q in range(2)]
            rhs = jnp.concatenate(
                [jnp.concatenate([xr[:, t[0]:t[0] + W],
                                  xr[:, t[1]:t[1] + W]], axis=1),
                 jnp.concatenate([xr[:, t[2]:t[2] + W],
                                  xr[:, t[3]:t[3] + W]], axis=1)],
                axis=0)
            g = jax.lax.dot_general(
                a_ref[...], rhs, (((1,), (0,)), ((), ())),
                preferred_element_type=jnp.float32
            ).astype(jnp.bfloat16)
            lhs = jnp.concatenate([g[:OC], g[OC:]], axis=1)
            d = jax.lax.dot_general(
                lhs, e_ref[...], (((1,), (0,)), ((), ())),
                preferred_element_type=jnp.float32)
            y = jnp.maximum(d * sc_ref[...] + sh_ref[...], 0.0)
            col = (2 * r + pl_) * W
            o_ref[0, :, col:col + W] = y


def kernel(x, conv_w, conv_b, bn_gamma, bn_beta, bn_mean, bn_var,
           eps=1e-5):
    N, C, H, W = x.shape
    assert H % 8 == 0 and W == 128, "specialized to W=128 lane tiles"
    H2, W2 = H // 2, W // 2
    out_ch = conv_w.shape[0]

    S = 0.5 * jnp.array([[1.,  1.,  1.,  1.],
                         [1.,  1., -1., -1.],
                         [1., -1.,  1., -1.],
                         [1., -1., -1.,  1.]], dtype=jnp.float32)
    w4 = conv_w.reshape(out_ch, 4, C).astype(jnp.float32)
    wc = jnp.einsum('bk,obc->koc', S, w4).reshape(2, 2, out_ch, C)

    scale = bn_gamma.astype(jnp.float32) / jnp.sqrt(
        bn_var.astype(jnp.float32) + eps)
    shift = (conv_b.astype(jnp.float32) - bn_mean.astype(jnp.float32)) \
        * scale + bn_beta.astype(jnp.float32)

    ocp = ((out_ch + 127) // 128) * 128
    pad = ocp - out_ch
    wc = jnp.pad(wc, ((0, 0), (0, 0), (0, pad), (0, 0)))
    scale = jnp.pad(scale, (0, pad))
    shift = jnp.pad(shift, (0, pad))

    A = wc.transpose(1, 2, 0, 3).reshape(2 * ocp, 2 * C) \
        .astype(jnp.bfloat16)

    En = np.zeros((2, 2 * W, W), np.float32)
    for q in range(2):
        for w2 in range(W2):
            En[0, q * W + 2 * w2, q * W2 + w2] = 1.0
            En[1, q * W + 2 * w2 + 1, q * W2 + w2] = 1.0
    E = jnp.asarray(En.reshape(4 * W, W), dtype=jnp.bfloat16)

    sc_mat = jnp.broadcast_to(scale[:, None], (ocp, W))
    sh_mat = jnp.broadcast_to(shift[:, None], (ocp, W))

    Bt = 4 if (H // 8) % 4 == 0 else 1
    T = H // 8 // Bt
    L = Bt * 2 * W

    out_flat = pl.pallas_call(
        _body,
        out_shape=jax.ShapeDtypeStruct((N, ocp, H2 * W2), x.dtype),
        grid=(N, T),
        in_specs=[
            pl.BlockSpec((1, C, Bt * 8, W), lambda n, t: (n, 0, t, 0)),
            pl.BlockSpec((2 * ocp, 2 * C), lambda n, t: (0, 0)),
            pl.BlockSpec((4 * W, W), lambda n, t: (0, 0)),
            pl.BlockSpec((ocp, W), lambda n, t: (0, 0)),
            pl.BlockSpec((ocp, W), lambda n, t: (0, 0)),
        ],
        out_specs=pl.BlockSpec((1, ocp, L), lambda n, t: (n, 0, t)),
        compiler_params=pltpu.CompilerParams(
            dimension_semantics=("parallel", "parallel"),
            vmem_limit_bytes=64 << 20),
    )(x, A, E, sc_mat, sh_mat)

    out = out_flat[:, :out_ch, :].reshape(N, out_ch, H2, W2)
    return out


# speedup vs baseline: 25.0848x; 1.2393x over previous
import numpy as np
import jax
import jax.numpy as jnp
from jax.experimental import pallas as pl
from jax.experimental.pallas import tpu as pltpu


def _body(x_ref, a_ref, e_ref, sc_ref, sh_ref, o_ref):
    _, C, Ht, W = x_ref.shape
    Bt = Ht // 8
    OC = a_ref.shape[0] // 2
    for r in range(Bt):
        xr = x_ref[0, :, 8 * r:8 * (r + 1), :] \
            .astype(jnp.bfloat16).reshape(C, 8 * W)
        for pl_ in range(2):
            t = [(4 * pl_ + 2 * q + hi) * W for hi in range(2)
                 for q in range(2)]
            xc = jnp.concatenate(
                [jnp.concatenate([xr[:, t[0]:t[0] + W],
                                  xr[:, t[1]:t[1] + W]], axis=1),
                 jnp.concatenate([xr[:, t[2]:t[2] + W],
                                  xr[:, t[3]:t[3] + W]], axis=1)],
                axis=0)
            g = jax.lax.dot_general(
                xc, a_ref[...], (((0,), (1,)), ((), ())),
                preferred_element_type=jnp.float32
            ).astype(jnp.bfloat16)
            lhs = jnp.concatenate([g[:, :OC], g[:, OC:]], axis=0)
            d = jax.lax.dot_general(
                e_ref[...], lhs, (((0,), (0,)), ((), ())),
                preferred_element_type=jnp.float32)
            y = jnp.maximum(d * sc_ref[...] + sh_ref[...], 0.0)
            row = (2 * r + pl_) * W
            o_ref[0, row:row + W, :] = y


def kernel(x, conv_w, conv_b, bn_gamma, bn_beta, bn_mean, bn_var,
           eps=1e-5):
    N, C, H, W = x.shape
    assert H % 8 == 0 and W == 128, "specialized to W=128 lane tiles"
    H2, W2 = H // 2, W // 2
    out_ch = conv_w.shape[0]

    S = 0.5 * jnp.array([[1.,  1.,  1.,  1.],
                         [1.,  1., -1., -1.],
                         [1., -1.,  1., -1.],
                         [1., -1., -1.,  1.]], dtype=jnp.float32)
    w4 = conv_w.reshape(out_ch, 4, C).astype(jnp.float32)
    wc = jnp.einsum('bk,obc->koc', S, w4).reshape(2, 2, out_ch, C)

    scale = bn_gamma.astype(jnp.float32) / jnp.sqrt(
        bn_var.astype(jnp.float32) + eps)
    shift = (conv_b.astype(jnp.float32) - bn_mean.astype(jnp.float32)) \
        * scale + bn_beta.astype(jnp.float32)

    ocp = ((out_ch + 127) // 128) * 128
    pad = ocp - out_ch
    wc = jnp.pad(wc, ((0, 0), (0, 0), (0, pad), (0, 0)))
    scale = jnp.pad(scale, (0, pad))
    shift = jnp.pad(shift, (0, pad))

    A = wc.transpose(1, 2, 0, 3).reshape(2 * ocp, 2 * C) \
        .astype(jnp.bfloat16)

    En = np.zeros((2, 2 * W, W), np.float32)
    for q in range(2):
        for w2 in range(W2):
            En[0, q * W + 2 * w2, q * W2 + w2] = 1.0
            En[1, q * W + 2 * w2 + 1, q * W2 + w2] = 1.0
    E = jnp.asarray(En.reshape(4 * W, W), dtype=jnp.bfloat16)

    sc_mat = scale.reshape(1, ocp)
    sh_mat = shift.reshape(1, ocp)

    Bt = 4 if (H // 8) % 4 == 0 else 1
    T = H // 8 // Bt
    L = Bt * 2 * W

    out_flat = pl.pallas_call(
        _body,
        out_shape=jax.ShapeDtypeStruct((N, H2 * W2, ocp), x.dtype),
        grid=(N, T),
        in_specs=[
            pl.BlockSpec((1, C, Bt * 8, W), lambda n, t: (n, 0, t, 0)),
            pl.BlockSpec((2 * ocp, 2 * C), lambda n, t: (0, 0)),
            pl.BlockSpec((4 * W, W), lambda n, t: (0, 0)),
            pl.BlockSpec((1, ocp), lambda n, t: (0, 0)),
            pl.BlockSpec((1, ocp), lambda n, t: (0, 0)),
        ],
        out_specs=pl.BlockSpec((1, L, ocp), lambda n, t: (n, t, 0)),
        compiler_params=pltpu.CompilerParams(
            dimension_semantics=("parallel", "parallel"),
            vmem_limit_bytes=64 << 20),
    )(x, A, E, sc_mat, sh_mat)

    out = out_flat[:, :, :out_ch].reshape(N, H2, W2, out_ch)
    return jnp.transpose(out, (0, 3, 1, 2))


# speedup vs baseline: 31.7863x; 1.2672x over previous
import jax
import jax.numpy as jnp
from jax.experimental import pallas as pl
from jax.experimental.pallas import tpu as pltpu


def _body(x_ref, a_ref, sc_ref, sh_ref, o_ref):
    _, C, Ht, W = x_ref.shape
    Bt = Ht // 8
    OC = a_ref.shape[0] // 2
    for r in range(Bt):
        xr = x_ref[0, :, 8 * r:8 * (r + 1), :] \
            .astype(jnp.bfloat16).reshape(C, 8 * W)
        for pl_ in range(2):
            t = [(4 * pl_ + 2 * q + hi) * W for hi in range(2)
                 for q in range(2)]
            xc = jnp.concatenate(
                [jnp.concatenate([xr[:, t[0]:t[0] + W],
                                  xr[:, t[1]:t[1] + W]], axis=1),
                 jnp.concatenate([xr[:, t[2]:t[2] + W],
                                  xr[:, t[3]:t[3] + W]], axis=1)],
                axis=0)
            g = jax.lax.dot_general(
                xc, a_ref[...], (((0,), (1,)), ((), ())),
                preferred_element_type=jnp.float32
            ).astype(jnp.bfloat16)
            g0 = pltpu.bitcast(g[:, :OC], jnp.int32)
            g1 = pltpu.bitcast(g[:, OC:], jnp.int32)
            lo = jax.lax.bitcast_convert_type(
                g0.astype(jnp.int16), jnp.bfloat16)
            hi = jax.lax.bitcast_convert_type(
                jax.lax.shift_right_logical(g1, 16).astype(jnp.int16),
                jnp.bfloat16)
            d = lo.astype(jnp.float32) + hi.astype(jnp.float32)
            y = jnp.maximum(d * sc_ref[...] + sh_ref[...], 0.0)
            row = (2 * r + pl_) * W
            o_ref[0, row:row + W, :] = y


def kernel(x, conv_w, conv_b, bn_gamma, bn_beta, bn_mean, bn_var,
           eps=1e-5):
    N, C, H, W = x.shape
    assert H % 8 == 0 and W == 128, "specialized to W=128 lane tiles"
    H2, W2 = H // 2, W // 2
    out_ch = conv_w.shape[0]

    S = 0.5 * jnp.array([[1.,  1.,  1.,  1.],
                         [1.,  1., -1., -1.],
                         [1., -1.,  1., -1.],
                         [1., -1., -1.,  1.]], dtype=jnp.float32)
    w4 = conv_w.reshape(out_ch, 4, C).astype(jnp.float32)
    wc = jnp.einsum('bk,obc->koc', S, w4).reshape(2, 2, out_ch, C)

    scale = bn_gamma.astype(jnp.float32) / jnp.sqrt(
        bn_var.astype(jnp.float32) + eps)
    shift = (conv_b.astype(jnp.float32) - bn_mean.astype(jnp.float32)) \
        * scale + bn_beta.astype(jnp.float32)

    ocp = ((out_ch + 127) // 128) * 128
    pad = ocp - out_ch
    wc = jnp.pad(wc, ((0, 0), (0, 0), (0, pad), (0, 0)))
    scale = jnp.pad(scale, (0, pad))
    shift = jnp.pad(shift, (0, pad))

    A = wc.transpose(1, 2, 0, 3).reshape(2 * ocp, 2 * C) \
        .astype(jnp.bfloat16)

    sc_mat = scale.reshape(1, ocp)
    sh_mat = shift.reshape(1, ocp)

    Bt = 4 if (H // 8) % 4 == 0 else 1
    T = H // 8 // Bt
    L = Bt * 2 * W

    out_flat = pl.pallas_call(
        _body,
        out_shape=jax.ShapeDtypeStruct((N, H2 * W2, ocp), x.dtype),
        grid=(N, T),
        in_specs=[
            pl.BlockSpec((1, C, Bt * 8, W), lambda n, t: (n, 0, t, 0)),
            pl.BlockSpec((2 * ocp, 2 * C), lambda n, t: (0, 0)),
            pl.BlockSpec((1, ocp), lambda n, t: (0, 0)),
            pl.BlockSpec((1, ocp), lambda n, t: (0, 0)),
        ],
        out_specs=pl.BlockSpec((1, L, ocp), lambda n, t: (n, t, 0)),
        compiler_params=pltpu.CompilerParams(
            dimension_semantics=("parallel", "parallel"),
            vmem_limit_bytes=64 << 20),
    )(x, A, sc_mat, sh_mat)

    out = out_flat[:, :, :out_ch].reshape(N, H2, W2, out_ch)
    return jnp.transpose(out, (0, 3, 1, 2))


# speedup vs baseline: 43.2732x; 1.3614x over previous
import jax
import jax.numpy as jnp
from jax.experimental import pallas as pl
from jax.experimental.pallas import tpu as pltpu


def _body(x_ref, a_ref, sc_ref, sh_ref, o_ref):
    _, C, Ht, W = x_ref.shape
    Bt = Ht // 8
    OC = a_ref.shape[0] // 2
    for r in range(Bt):
        xr = x_ref[0, :, 8 * r:8 * (r + 1), :] \
            .astype(jnp.bfloat16).reshape(C, 8 * W)
        for pl_ in range(2):
            t = [(4 * pl_ + 2 * q + hi) * W for hi in range(2)
                 for q in range(2)]
            xc = jnp.concatenate(
                [jnp.concatenate([xr[:, t[0]:t[0] + W],
                                  xr[:, t[1]:t[1] + W]], axis=1),
                 jnp.concatenate([xr[:, t[2]:t[2] + W],
                                  xr[:, t[3]:t[3] + W]], axis=1)],
                axis=0)
            g = jax.lax.dot_general(
                xc, a_ref[...], (((0,), (1,)), ((), ())),
                preferred_element_type=jnp.float32
            ).astype(jnp.bfloat16)
            g0 = pltpu.bitcast(g[:, :OC], jnp.int32)
            g1 = pltpu.bitcast(g[:, OC:], jnp.int32)
            lo = jax.lax.bitcast_convert_type(
                g0.astype(jnp.int16), jnp.bfloat16)
            hi = jax.lax.bitcast_convert_type(
                jax.lax.shift_right_logical(g1, 16).astype(jnp.int16),
                jnp.bfloat16)
            d = lo.astype(jnp.float32) + hi.astype(jnp.float32)
            y = jnp.maximum(d * sc_ref[...] + sh_ref[...], 0.0)
            row = (2 * r + pl_) * W
            o_ref[0, row:row + W, :] = y


def kernel(x, conv_w, conv_b, bn_gamma, bn_beta, bn_mean, bn_var,
           eps=1e-5):
    N, C, H, W = x.shape
    assert H % 8 == 0 and W == 128, "specialized to W=128 lane tiles"
    H2, W2 = H // 2, W // 2
    out_ch = conv_w.shape[0]

    S = 0.5 * jnp.array([[1.,  1.,  1.,  1.],
                         [1.,  1., -1., -1.],
                         [1., -1.,  1., -1.],
                         [1., -1., -1.,  1.]], dtype=jnp.float32)
    w4 = conv_w.reshape(out_ch, 4, C).astype(jnp.float32)
    wc = jnp.einsum('bk,obc->koc', S, w4).reshape(2, 2, out_ch, C)

    scale = bn_gamma.astype(jnp.float32) / jnp.sqrt(
        bn_var.astype(jnp.float32) + eps)
    shift = (conv_b.astype(jnp.float32) - bn_mean.astype(jnp.float32)) \
        * scale + bn_beta.astype(jnp.float32)

    ocp = ((out_ch + 127) // 128) * 128
    pad = ocp - out_ch
    wc = jnp.pad(wc, ((0, 0), (0, 0), (0, pad), (0, 0)))
    scale = jnp.pad(scale, (0, pad))
    shift = jnp.pad(shift, (0, pad))

    A = wc.transpose(1, 2, 0, 3).reshape(2 * ocp, 2 * C) \
        .astype(jnp.bfloat16)

    sc_mat = scale.reshape(1, ocp)
    sh_mat = shift.reshape(1, ocp)

    Bt = 8 if (H // 8) % 8 == 0 else (4 if (H // 8) % 4 == 0 else 1)
    T = H // 8 // Bt
    L = Bt * 2 * W

    out_flat = pl.pallas_call(
        _body,
        out_shape=jax.ShapeDtypeStruct((N, H2 * W2, ocp), x.dtype),
        grid=(N, T),
        in_specs=[
            pl.BlockSpec((1, C, Bt * 8, W), lambda n, t: (n, 0, t, 0)),
            pl.BlockSpec((2 * ocp, 2 * C), lambda n, t: (0, 0)),
            pl.BlockSpec((1, ocp), lambda n, t: (0, 0)),
            pl.BlockSpec((1, ocp), lambda n, t: (0, 0)),
        ],
        out_specs=pl.BlockSpec((1, L, ocp), lambda n, t: (n, t, 0)),
        compiler_params=pltpu.CompilerParams(
            dimension_semantics=("parallel", "parallel"),
            vmem_limit_bytes=64 << 20),
    )(x, A, sc_mat, sh_mat)

    out = out_flat[:, :, :out_ch].reshape(N, H2, W2, out_ch)
    return jnp.transpose(out, (0, 3, 1, 2))


# speedup vs baseline: 53.4874x; 1.2360x over previous
import jax
import jax.numpy as jnp
from jax.experimental import pallas as pl
from jax.experimental.pallas import tpu as pltpu


def _body(x_ref, a_ref, sc_ref, sh_ref, o_ref):
    _, C, Ht, W = x_ref.shape
    Bt = Ht // 8
    OC = a_ref.shape[0] // 2
    for r in range(Bt):
        xr = x_ref[0, :, 8 * r:8 * (r + 1), :] \
            .astype(jnp.bfloat16).reshape(C, 8 * W)
        for pl_ in range(2):
            t = [(4 * pl_ + 2 * q + hi) * W for hi in range(2)
                 for q in range(2)]
            xc = jnp.concatenate(
                [jnp.concatenate([xr[:, t[0]:t[0] + W],
                                  xr[:, t[1]:t[1] + W]], axis=1),
                 jnp.concatenate([xr[:, t[2]:t[2] + W],
                                  xr[:, t[3]:t[3] + W]], axis=1)],
                axis=0)
            g = jax.lax.dot_general(
                xc, a_ref[...], (((0,), (1,)), ((), ())),
                preferred_element_type=jnp.float32
            ).astype(jnp.bfloat16)
            g0 = pltpu.bitcast(g[:, :OC], jnp.int32)
            g1 = pltpu.bitcast(g[:, OC:], jnp.int32)
            lo = jax.lax.bitcast_convert_type(
                g0.astype(jnp.int16), jnp.bfloat16)
            hi = jax.lax.bitcast_convert_type(
                jax.lax.shift_right_logical(g1, 16).astype(jnp.int16),
                jnp.bfloat16)
            d = lo.astype(jnp.float32) + hi.astype(jnp.float32)
            y = jnp.maximum(d * sc_ref[...] + sh_ref[...], 0.0)
            row = (2 * r + pl_) * W
            o_ref[0, row:row + W, :] = y


def kernel(x, conv_w, conv_b, bn_gamma, bn_beta, bn_mean, bn_var,
           eps=1e-5):
    N, C, H, W = x.shape
    assert H % 8 == 0 and W == 128, "specialized to W=128 lane tiles"
    H2, W2 = H // 2, W // 2
    out_ch = conv_w.shape[0]

    S = 0.5 * jnp.array([[1.,  1.,  1.,  1.],
                         [1.,  1., -1., -1.],
                         [1., -1.,  1., -1.],
                         [1., -1., -1.,  1.]], dtype=jnp.float32)
    w4 = conv_w.reshape(out_ch, 4, C).astype(jnp.float32)
    wc = jnp.einsum('bk,obc->koc', S, w4).reshape(2, 2, out_ch, C)

    scale = bn_gamma.astype(jnp.float32) / jnp.sqrt(
        bn_var.astype(jnp.float32) + eps)
    shift = (conv_b.astype(jnp.float32) - bn_mean.astype(jnp.float32)) \
        * scale + bn_beta.astype(jnp.float32)

    ocp = ((out_ch + 127) // 128) * 128
    pad = ocp - out_ch
    wc = jnp.pad(wc, ((0, 0), (0, 0), (0, pad), (0, 0)))
    scale = jnp.pad(scale, (0, pad))
    shift = jnp.pad(shift, (0, pad))

    A = wc.transpose(1, 2, 0, 3).reshape(2 * ocp, 2 * C) \
        .astype(jnp.bfloat16)

    sc_mat = scale.reshape(1, ocp)
    sh_mat = shift.reshape(1, ocp)

    Bt = 16 if (H // 8) % 16 == 0 else (
        8 if (H // 8) % 8 == 0 else (4 if (H // 8) % 4 == 0 else 1))
    T = H // 8 // Bt
    L = Bt * 2 * W

    out_flat = pl.pallas_call(
        _body,
        out_shape=jax.ShapeDtypeStruct((N, H2 * W2, ocp), x.dtype),
        grid=(N, T),
        in_specs=[
            pl.BlockSpec((1, C, Bt * 8, W), lambda n, t: (n, 0, t, 0)),
            pl.BlockSpec((2 * ocp, 2 * C), lambda n, t: (0, 0)),
            pl.BlockSpec((1, ocp), lambda n, t: (0, 0)),
            pl.BlockSpec((1, ocp), lambda n, t: (0, 0)),
        ],
        out_specs=pl.BlockSpec((1, L, ocp), lambda n, t: (n, t, 0)),
        compiler_params=pltpu.CompilerParams(
            dimension_semantics=("parallel", "parallel"),
            vmem_limit_bytes=64 << 20),
    )(x, A, sc_mat, sh_mat)

    out = out_flat[:, :, :out_ch].reshape(N, H2, W2, out_ch)
    return jnp.transpose(out, (0, 3, 1, 2))


# speedup vs baseline: 56.2542x; 1.0517x over previous
import jax
import jax.numpy as jnp
from jax.experimental import pallas as pl
from jax.experimental.pallas import tpu as pltpu


def _body(x_ref, a_ref, sc_ref, sh_ref, o_ref):
    NB, C, Ht, W = x_ref.shape
    Bt = Ht // 8
    OC = a_ref.shape[0] // 2
    for nn in range(NB):
      for r in range(Bt):
        xr = x_ref[nn, :, 8 * r:8 * (r + 1), :] \
            .astype(jnp.bfloat16).reshape(C, 8 * W)
        for pl_ in range(2):
            t = [(4 * pl_ + 2 * q + hi) * W for hi in range(2)
                 for q in range(2)]
            xc = jnp.concatenate(
                [jnp.concatenate([xr[:, t[0]:t[0] + W],
                                  xr[:, t[1]:t[1] + W]], axis=1),
                 jnp.concatenate([xr[:, t[2]:t[2] + W],
                                  xr[:, t[3]:t[3] + W]], axis=1)],
                axis=0)
            g = jax.lax.dot_general(
                xc, a_ref[...], (((0,), (1,)), ((), ())),
                preferred_element_type=jnp.float32
            ).astype(jnp.bfloat16)
            g0 = pltpu.bitcast(g[:, :OC], jnp.int32)
            g1 = pltpu.bitcast(g[:, OC:], jnp.int32)
            lo = jax.lax.bitcast_convert_type(
                g0.astype(jnp.int16), jnp.bfloat16)
            hi = jax.lax.bitcast_convert_type(
                jax.lax.shift_right_logical(g1, 16).astype(jnp.int16),
                jnp.bfloat16)
            d = lo.astype(jnp.float32) + hi.astype(jnp.float32)
            y = jnp.maximum(d * sc_ref[...] + sh_ref[...], 0.0)
            row = (2 * r + pl_) * W
            o_ref[nn, row:row + W, :] = y


def kernel(x, conv_w, conv_b, bn_gamma, bn_beta, bn_mean, bn_var,
           eps=1e-5):
    N, C, H, W = x.shape
    assert H % 8 == 0 and W == 128, "specialized to W=128 lane tiles"
    H2, W2 = H // 2, W // 2
    out_ch = conv_w.shape[0]

    S = 0.5 * jnp.array([[1.,  1.,  1.,  1.],
                         [1.,  1., -1., -1.],
                         [1., -1.,  1., -1.],
                         [1., -1., -1.,  1.]], dtype=jnp.float32)
    w4 = conv_w.reshape(out_ch, 4, C).astype(jnp.float32)
    wc = jnp.einsum('bk,obc->koc', S, w4).reshape(2, 2, out_ch, C)

    scale = bn_gamma.astype(jnp.float32) / jnp.sqrt(
        bn_var.astype(jnp.float32) + eps)
    shift = (conv_b.astype(jnp.float32) - bn_mean.astype(jnp.float32)) \
        * scale + bn_beta.astype(jnp.float32)

    ocp = ((out_ch + 127) // 128) * 128
    pad = ocp - out_ch
    wc = jnp.pad(wc, ((0, 0), (0, 0), (0, pad), (0, 0)))
    scale = jnp.pad(scale, (0, pad))
    shift = jnp.pad(shift, (0, pad))

    A = wc.transpose(1, 2, 0, 3).reshape(2 * ocp, 2 * C) \
        .astype(jnp.bfloat16)

    sc_mat = scale.reshape(1, ocp)
    sh_mat = shift.reshape(1, ocp)

    Bt = 16 if (H // 8) % 16 == 0 else (
        8 if (H // 8) % 8 == 0 else (4 if (H // 8) % 4 == 0 else 1))
    T = H // 8 // Bt
    L = Bt * 2 * W

    NB = 2 if N % 2 == 0 else 1

    out_flat = pl.pallas_call(
        _body,
        out_shape=jax.ShapeDtypeStruct((N, H2 * W2, ocp), x.dtype),
        grid=(N // NB, T),
        in_specs=[
            pl.BlockSpec((NB, C, Bt * 8, W), lambda n, t: (n, 0, t, 0)),
            pl.BlockSpec((2 * ocp, 2 * C), lambda n, t: (0, 0)),
            pl.BlockSpec((1, ocp), lambda n, t: (0, 0)),
            pl.BlockSpec((1, ocp), lambda n, t: (0, 0)),
        ],
        out_specs=pl.BlockSpec((NB, L, ocp), lambda n, t: (n, t, 0)),
        compiler_params=pltpu.CompilerParams(
            dimension_semantics=("parallel", "parallel"),
            vmem_limit_bytes=64 << 20),
    )(x, A, sc_mat, sh_mat)

    out = out_flat[:, :, :out_ch].reshape(N, H2, W2, out_ch)
    return jnp.transpose(out, (0, 3, 1, 2))
